# Initial kernel scaffold; baseline (speedup 1.0000x reference)
import jax
import jax.numpy as jnp
from jax.experimental import pallas as pl


def kernel(z, codebook, in_v, in_g, in_b, out_v, out_g, out_b):
    raise NotImplementedError("write your pallas kernel here")


# speedup vs baseline: 1.0010x; 1.0010x over previous
import functools

import jax
import jax.numpy as jnp
from jax import lax
from jax.experimental import pallas as pl
from jax.experimental.pallas import tpu as pltpu
from jax.experimental.pallas import tpu_sc as plsc

B, CIN, T = 8, 768, 1024
K, DC = 8192, 64
TBLK = 512
KBLK = 2048
NT = T // TBLK
NTB = (B * T) // TBLK
NK = K // KBLK


def _stageA_body(z_ref, w_ref, b_ref, ze_ref):
    zb = z_ref[0].astype(jnp.bfloat16)
    ze = lax.dot_general(w_ref[...], zb, (((1,), (0,)), ((), ())),
                         preferred_element_type=jnp.float32)
    ze_ref[0] = ze + b_ref[...]


_STAGEA_KW = dict(
    grid=(B, NT),
    in_specs=[
        pl.BlockSpec((1, CIN, TBLK), lambda b, t: (b, 0, t)),
        pl.BlockSpec((DC, CIN), lambda b, t: (0, 0)),
        pl.BlockSpec((DC, 1), lambda b, t: (0, 0)),
    ],
    out_specs=pl.BlockSpec((1, DC, TBLK), lambda b, t: (b, 0, t)),
    out_shape=jax.ShapeDtypeStruct((B, DC, T), jnp.float32),
)

_stageA = pl.pallas_call(_stageA_body, **_STAGEA_KW)


def _stageB_body(encn_ref, cbn_ref, r_ref, c_ref, idx_ref, best_ref, bidx_ref):
    k = pl.program_id(1)

    @pl.when(k == 0)
    def _():
        best_ref[...] = jnp.full((TBLK, 1), -jnp.inf, jnp.float32)
        bidx_ref[...] = jnp.zeros((TBLK, 1), jnp.int32)

    s = lax.dot_general(encn_ref[...].astype(jnp.bfloat16),
                        cbn_ref[pl.ds(k * KBLK, KBLK), :].astype(jnp.bfloat16),
                        (((1,), (1,)), ((), ())),
                        preferred_element_type=jnp.float32)
    dist = (r_ref[...] - 2.0 * s) + c_ref[...]
    minval = jnp.min(dist, axis=1, keepdims=True)
    iota = lax.broadcasted_iota(jnp.int32, (TBLK, KBLK), 1) + k * KBLK
    minidx = jnp.min(jnp.where(dist == minval, iota, K),
                     axis=1, keepdims=True)
    cand = -minval
    better = cand > best_ref[...]
    new_best = jnp.where(
        better, cand.astype(jnp.bfloat16).astype(jnp.float32), best_ref[...])
    new_bidx = jnp.where(better, minidx, bidx_ref[...])
    best_ref[...] = new_best
    bidx_ref[...] = new_bidx

    @pl.when(k == NK - 1)
    def _():
        idx_ref[0] = new_bidx


_STAGEB_KW = dict(
    grid=(NTB, NK),
    in_specs=[
        pl.BlockSpec((TBLK, DC), lambda t, k: (t, 0)),
        pl.BlockSpec((K, DC), lambda t, k: (0, 0)),
        pl.BlockSpec((TBLK, 1), lambda t, k: (t, 0)),
        pl.BlockSpec((1, KBLK), lambda t, k: (0, k)),
    ],
    out_specs=pl.BlockSpec((1, TBLK, 1), lambda t, k: (t, 0, 0)),
    out_shape=jax.ShapeDtypeStruct((NTB, TBLK, 1), jnp.int32),
    scratch_shapes=[
        pltpu.VMEM((TBLK, 1), jnp.float32),
        pltpu.VMEM((TBLK, 1), jnp.int32),
    ],
)

_stageB = pl.pallas_call(_stageB_body, **_STAGEB_KW)


def _stageD_body(zq_ref, ze_ref, w_ref, b_ref, out_ref, loss_ref):
    zq = zq_ref[0][:, :DC]
    o = lax.dot_general(w_ref[...], zq.astype(jnp.bfloat16),
                        (((1,), (1,)), ((), ())),
                        preferred_element_type=jnp.float32)
    out_ref[0] = o + b_ref[...]
    diff = ze_ref[0] - zq.T
    msq = jnp.sum(diff * diff) * (1.0 / (DC * T))
    loss_ref[...] = jnp.full((1, 1, 128), msq, jnp.float32)


_STAGED_KW = dict(
    grid=(B,),
    in_specs=[
        pl.BlockSpec((1, T, 128), lambda b: (b, 0, 0)),
        pl.BlockSpec((1, DC, T), lambda b: (b, 0, 0)),
        pl.BlockSpec((CIN, DC), lambda b: (0, 0)),
        pl.BlockSpec((CIN, 1), lambda b: (0, 0)),
    ],
    out_specs=[
        pl.BlockSpec((1, CIN, T), lambda b: (b, 0, 0)),
        pl.BlockSpec((1, 1, 128), lambda b: (b, 0, 0)),
    ],
    out_shape=[
        jax.ShapeDtypeStruct((B, CIN, T), jnp.float32),
        jax.ShapeDtypeStruct((B, 1, 128), jnp.float32),
    ],
)

_stageD = pl.pallas_call(_stageD_body, **_STAGED_KW)

_NC, _NS = 2, 16
_NW = _NC * _NS
_BPW = (B * T) // _NW
_CHUNK = 128


def _sc_gather_body(cb_hbm, idx_hbm, out_hbm, idx_a, idx_b, rows_a, rows_b,
                    sem):
    wid = lax.axis_index("s") * _NC + lax.axis_index("c")
    base = wid * _BPW
    pltpu.sync_copy(idx_hbm.at[pl.ds(base, _CHUNK)], idx_a)
    pltpu.sync_copy(idx_hbm.at[pl.ds(base + _CHUNK, _CHUNK)], idx_b)
    ca = pltpu.async_copy(cb_hbm.at[idx_a], rows_a, sem)
    cb = pltpu.async_copy(cb_hbm.at[idx_b], rows_b, sem)
    ca.wait()
    cb.wait()
    pltpu.sync_copy(rows_a, out_hbm.at[pl.ds(base, _CHUNK)])
    pltpu.sync_copy(rows_b, out_hbm.at[pl.ds(base + _CHUNK, _CHUNK)])


@functools.cache
def _sc_gather():
    mesh = plsc.VectorSubcoreMesh(core_axis_name="c", subcore_axis_name="s")
    return pl.kernel(
        _sc_gather_body,
        mesh=mesh,
        out_type=jax.ShapeDtypeStruct((B * T, 128), jnp.float32),
        scratch_types=[
            pltpu.VMEM((_CHUNK,), jnp.int32),
            pltpu.VMEM((_CHUNK,), jnp.int32),
            pltpu.VMEM((_CHUNK, 128), jnp.float32),
            pltpu.VMEM((_CHUNK, 128), jnp.float32),
            pltpu.SemaphoreType.DMA,
        ],
    )


def kernel(z, codebook, in_v, in_g, in_b, out_v, out_g, out_b):
    f32 = jnp.float32
    bf16 = jnp.bfloat16
    w_in = in_g * in_v / jnp.maximum(
        jnp.sqrt(jnp.sum(in_v * in_v, axis=1, keepdims=True)), 1e-12)
    w_out = out_g * out_v / jnp.maximum(
        jnp.sqrt(jnp.sum(out_v * out_v, axis=1, keepdims=True)), 1e-12)
    cb_n = codebook / jnp.maximum(
        jnp.sqrt(jnp.sum(codebook * codebook, axis=1, keepdims=True)), 1e-12)
    c = jnp.sum(cb_n ** 2, axis=1, keepdims=True).T

    ze = _stageA(z, w_in.astype(bf16), in_b.reshape(DC, 1))

    enc = jnp.transpose(ze, (0, 2, 1)).reshape(B * T, DC)
    n = jnp.sqrt(jnp.sum(enc * enc, axis=-1, keepdims=True))
    encn = enc / jnp.maximum(n, 1e-12)
    r = jnp.sum(encn ** 2, axis=1, keepdims=True)

    idx3 = _stageB(encn, cb_n, r, c)
    idx_flat = idx3.reshape(B * T)

    cb_pad = jnp.concatenate(
        [codebook, jnp.zeros((K, 128 - DC), f32)], axis=1)
    zq_flat = _sc_gather()(cb_pad, idx_flat)
    zq = zq_flat.reshape(B, T, 128)

    out, loss3 = _stageD(zq, ze, w_out.astype(bf16), out_b.reshape(CIN, 1))
    loss = loss3[:, 0, 0]
    indices = idx_flat.reshape(B, T)
    return (out, loss, loss, indices, ze)


# speedup vs baseline: 1.0843x; 1.0831x over previous
import functools

import jax
import jax.numpy as jnp
from jax import lax
from jax.experimental import pallas as pl
from jax.experimental.pallas import tpu as pltpu
from jax.experimental.pallas import tpu_sc as plsc

B, CIN, T = 8, 768, 1024
K, DC = 8192, 64
TBLK = 1024
KBLK = 2048
NT = 1
NTB = (B * T) // TBLK
NK = K // KBLK


def _stageA_body(z_ref, w_ref, b_ref, ze_ref):
    zb = z_ref[0].astype(jnp.bfloat16)
    ze = lax.dot_general(w_ref[...], zb, (((1,), (0,)), ((), ())),
                         preferred_element_type=jnp.float32)
    ze_ref[0] = ze + b_ref[...]


_STAGEA_KW = dict(
    grid=(B,),
    in_specs=[
        pl.BlockSpec((1, CIN, T), lambda b: (b, 0, 0)),
        pl.BlockSpec((DC, CIN), lambda b: (0, 0)),
        pl.BlockSpec((DC, 1), lambda b: (0, 0)),
    ],
    out_specs=pl.BlockSpec((1, DC, T), lambda b: (b, 0, 0)),
    out_shape=jax.ShapeDtypeStruct((B, DC, T), jnp.float32),
)

_stageA = pl.pallas_call(_stageA_body, **_STAGEA_KW)


def _stageB_body(encn_ref, cbn_ref, r_ref, c_ref, idx_ref, best_ref, bidx_ref):
    k = pl.program_id(1)

    @pl.when(k == 0)
    def _():
        best_ref[...] = jnp.full((TBLK, 1), -jnp.inf, jnp.float32)
        bidx_ref[...] = jnp.zeros((TBLK, 1), jnp.int32)

    s = lax.dot_general(encn_ref[...].astype(jnp.bfloat16),
                        cbn_ref[pl.ds(k * KBLK, KBLK), :].astype(jnp.bfloat16),
                        (((1,), (1,)), ((), ())),
                        preferred_element_type=jnp.float32)
    dist = (r_ref[...] - 2.0 * s) + c_ref[...]
    minval = jnp.min(dist, axis=1, keepdims=True)
    iota = lax.broadcasted_iota(jnp.int32, (TBLK, KBLK), 1) + k * KBLK
    minidx = jnp.min(jnp.where(dist == minval, iota, K),
                     axis=1, keepdims=True)
    cand = -minval
    better = cand > best_ref[...]
    new_best = jnp.where(
        better, cand.astype(jnp.bfloat16).astype(jnp.float32), best_ref[...])
    new_bidx = jnp.where(better, minidx, bidx_ref[...])
    best_ref[...] = new_best
    bidx_ref[...] = new_bidx

    @pl.when(k == NK - 1)
    def _():
        idx_ref[0] = new_bidx


_STAGEB_KW = dict(
    grid=(NTB, NK),
    in_specs=[
        pl.BlockSpec((TBLK, DC), lambda t, k: (t, 0)),
        pl.BlockSpec((K, DC), lambda t, k: (0, 0)),
        pl.BlockSpec((TBLK, 1), lambda t, k: (t, 0)),
        pl.BlockSpec((1, KBLK), lambda t, k: (0, k)),
    ],
    out_specs=pl.BlockSpec((1, TBLK, 1), lambda t, k: (t, 0, 0)),
    out_shape=jax.ShapeDtypeStruct((NTB, TBLK, 1), jnp.int32),
    scratch_shapes=[
        pltpu.VMEM((TBLK, 1), jnp.float32),
        pltpu.VMEM((TBLK, 1), jnp.int32),
    ],
)

_stageB = pl.pallas_call(_stageB_body, **_STAGEB_KW)


def _stageD_body(zq_ref, ze_ref, w_ref, b_ref, out_ref, loss_ref):
    zq = zq_ref[0][:, :DC]
    o = lax.dot_general(w_ref[...], zq.astype(jnp.bfloat16),
                        (((1,), (1,)), ((), ())),
                        preferred_element_type=jnp.float32)
    out_ref[0] = o + b_ref[...]
    diff = ze_ref[0] - zq.T
    msq = jnp.sum(diff * diff) * (1.0 / (DC * T))
    loss_ref[...] = jnp.full((1, 1, 128), msq, jnp.float32)


_STAGED_KW = dict(
    grid=(B,),
    in_specs=[
        pl.BlockSpec((1, T, 128), lambda b: (b, 0, 0)),
        pl.BlockSpec((1, DC, T), lambda b: (b, 0, 0)),
        pl.BlockSpec((CIN, DC), lambda b: (0, 0)),
        pl.BlockSpec((CIN, 1), lambda b: (0, 0)),
    ],
    out_specs=[
        pl.BlockSpec((1, CIN, T), lambda b: (b, 0, 0)),
        pl.BlockSpec((1, 1, 128), lambda b: (b, 0, 0)),
    ],
    out_shape=[
        jax.ShapeDtypeStruct((B, CIN, T), jnp.float32),
        jax.ShapeDtypeStruct((B, 1, 128), jnp.float32),
    ],
)

_stageD = pl.pallas_call(_stageD_body, **_STAGED_KW)

_NC, _NS = 2, 16
_NW = _NC * _NS
_BPW = (B * T) // _NW
_CHUNK = 128


def _sc_gather_body(cb_hbm, idx_hbm, out_hbm, idx_a, idx_b, rows_a, rows_b,
                    sem):
    wid = lax.axis_index("s") * _NC + lax.axis_index("c")
    base = wid * _BPW
    pltpu.sync_copy(idx_hbm.at[pl.ds(base, _CHUNK)], idx_a)
    pltpu.sync_copy(idx_hbm.at[pl.ds(base + _CHUNK, _CHUNK)], idx_b)
    ca = pltpu.async_copy(cb_hbm.at[idx_a], rows_a, sem)
    cb = pltpu.async_copy(cb_hbm.at[idx_b], rows_b, sem)
    ca.wait()
    cb.wait()
    pltpu.sync_copy(rows_a, out_hbm.at[pl.ds(base, _CHUNK)])
    pltpu.sync_copy(rows_b, out_hbm.at[pl.ds(base + _CHUNK, _CHUNK)])


@functools.cache
def _sc_gather():
    mesh = plsc.VectorSubcoreMesh(core_axis_name="c", subcore_axis_name="s")
    return pl.kernel(
        _sc_gather_body,
        mesh=mesh,
        out_type=jax.ShapeDtypeStruct((B * T, 128), jnp.float32),
        scratch_types=[
            pltpu.VMEM((_CHUNK,), jnp.int32),
            pltpu.VMEM((_CHUNK,), jnp.int32),
            pltpu.VMEM((_CHUNK, 128), jnp.float32),
            pltpu.VMEM((_CHUNK, 128), jnp.float32),
            pltpu.SemaphoreType.DMA,
        ],
    )


def kernel(z, codebook, in_v, in_g, in_b, out_v, out_g, out_b):
    f32 = jnp.float32
    bf16 = jnp.bfloat16
    w_in = in_g * in_v / jnp.maximum(
        jnp.sqrt(jnp.sum(in_v * in_v, axis=1, keepdims=True)), 1e-12)
    w_out = out_g * out_v / jnp.maximum(
        jnp.sqrt(jnp.sum(out_v * out_v, axis=1, keepdims=True)), 1e-12)
    cb_n = codebook / jnp.maximum(
        jnp.sqrt(jnp.sum(codebook * codebook, axis=1, keepdims=True)), 1e-12)
    c = jnp.sum(cb_n ** 2, axis=1, keepdims=True).T

    ze = _stageA(z, w_in.astype(bf16), in_b.reshape(DC, 1))

    enc = jnp.transpose(ze, (0, 2, 1)).reshape(B * T, DC)
    n = jnp.sqrt(jnp.sum(enc * enc, axis=-1, keepdims=True))
    encn = enc / jnp.maximum(n, 1e-12)
    r = jnp.sum(encn ** 2, axis=1, keepdims=True)

    idx3 = _stageB(encn, cb_n, r, c)
    idx_flat = idx3.reshape(B * T)

    cb_pad = jnp.concatenate(
        [codebook, jnp.zeros((K, 128 - DC), f32)], axis=1)
    zq_flat = _sc_gather()(cb_pad, idx_flat)
    zq = zq_flat.reshape(B, T, 128)

    out, loss3 = _stageD(zq, ze, w_out.astype(bf16), out_b.reshape(CIN, 1))
    loss = loss3[:, 0, 0]
    indices = idx_flat.reshape(B, T)
    return (out, loss, loss, indices, ze)


# speedup vs baseline: 1.1033x; 1.0176x over previous
import functools

import jax
import jax.numpy as jnp
from jax import lax
from jax.experimental import pallas as pl
from jax.experimental.pallas import tpu as pltpu
from jax.experimental.pallas import tpu_sc as plsc

B, CIN, T = 8, 768, 1024
K, DC = 8192, 64
TBLK = 2048
KBLK = 2048
NT = 1
NTB = (B * T) // TBLK
NK = K // KBLK


def _stageA_body(z_ref, w_ref, b_ref, ze_ref):
    zb = z_ref[0].astype(jnp.bfloat16)
    ze = lax.dot_general(w_ref[...], zb, (((1,), (0,)), ((), ())),
                         preferred_element_type=jnp.float32)
    ze_ref[0] = ze + b_ref[...]


_STAGEA_KW = dict(
    grid=(B,),
    in_specs=[
        pl.BlockSpec((1, CIN, T), lambda b: (b, 0, 0)),
        pl.BlockSpec((DC, CIN), lambda b: (0, 0)),
        pl.BlockSpec((DC, 1), lambda b: (0, 0)),
    ],
    out_specs=pl.BlockSpec((1, DC, T), lambda b: (b, 0, 0)),
    out_shape=jax.ShapeDtypeStruct((B, DC, T), jnp.float32),
)

_stageA = pl.pallas_call(_stageA_body, **_STAGEA_KW)


def _stageB_body(encn_ref, cbn_ref, r_ref, c_ref, idx_ref, best_ref, bidx_ref):
    k = pl.program_id(1)

    @pl.when(k == 0)
    def _():
        best_ref[...] = jnp.full((TBLK, 1), -jnp.inf, jnp.float32)
        bidx_ref[...] = jnp.zeros((TBLK, 1), jnp.int32)

    s = lax.dot_general(encn_ref[...].astype(jnp.bfloat16),
                        cbn_ref[pl.ds(k * KBLK, KBLK), :].astype(jnp.bfloat16),
                        (((1,), (1,)), ((), ())),
                        preferred_element_type=jnp.float32)
    dist = (r_ref[...] - 2.0 * s) + c_ref[...]
    minval = jnp.min(dist, axis=1, keepdims=True)
    iota = lax.broadcasted_iota(jnp.int32, (TBLK, KBLK), 1) + k * KBLK
    minidx = jnp.min(jnp.where(dist == minval, iota, K),
                     axis=1, keepdims=True)
    cand = -minval
    better = cand > best_ref[...]
    new_best = jnp.where(
        better, cand.astype(jnp.bfloat16).astype(jnp.float32), best_ref[...])
    new_bidx = jnp.where(better, minidx, bidx_ref[...])
    best_ref[...] = new_best
    bidx_ref[...] = new_bidx

    @pl.when(k == NK - 1)
    def _():
        idx_ref[0] = new_bidx


_STAGEB_KW = dict(
    grid=(NTB, NK),
    in_specs=[
        pl.BlockSpec((TBLK, DC), lambda t, k: (t, 0)),
        pl.BlockSpec((K, DC), lambda t, k: (0, 0)),
        pl.BlockSpec((TBLK, 1), lambda t, k: (t, 0)),
        pl.BlockSpec((1, KBLK), lambda t, k: (0, k)),
    ],
    out_specs=pl.BlockSpec((1, TBLK, 1), lambda t, k: (t, 0, 0)),
    out_shape=jax.ShapeDtypeStruct((NTB, TBLK, 1), jnp.int32),
    scratch_shapes=[
        pltpu.VMEM((TBLK, 1), jnp.float32),
        pltpu.VMEM((TBLK, 1), jnp.int32),
    ],
)

_stageB = pl.pallas_call(_stageB_body, **_STAGEB_KW)


def _stageD_body(zq_ref, ze_ref, w_ref, b_ref, out_ref, loss_ref):
    zq = zq_ref[0][:, :DC]
    o = lax.dot_general(w_ref[...], zq.astype(jnp.bfloat16),
                        (((1,), (1,)), ((), ())),
                        preferred_element_type=jnp.float32)
    out_ref[0] = o + b_ref[...]
    diff = ze_ref[0] - zq.T
    msq = jnp.sum(diff * diff) * (1.0 / (DC * T))
    loss_ref[...] = jnp.full((1, 1, 128), msq, jnp.float32)


_STAGED_KW = dict(
    grid=(B,),
    in_specs=[
        pl.BlockSpec((1, T, 128), lambda b: (b, 0, 0)),
        pl.BlockSpec((1, DC, T), lambda b: (b, 0, 0)),
        pl.BlockSpec((CIN, DC), lambda b: (0, 0)),
        pl.BlockSpec((CIN, 1), lambda b: (0, 0)),
    ],
    out_specs=[
        pl.BlockSpec((1, CIN, T), lambda b: (b, 0, 0)),
        pl.BlockSpec((1, 1, 128), lambda b: (b, 0, 0)),
    ],
    out_shape=[
        jax.ShapeDtypeStruct((B, CIN, T), jnp.float32),
        jax.ShapeDtypeStruct((B, 1, 128), jnp.float32),
    ],
)

_stageD = pl.pallas_call(_stageD_body, **_STAGED_KW)

_NC, _NS = 2, 16
_NW = _NC * _NS
_BPW = (B * T) // _NW
_CHUNK = 128


def _sc_gather_body(cb_hbm, idx_hbm, out_hbm, idx_a, idx_b, rows_a, rows_b,
                    sem):
    wid = lax.axis_index("s") * _NC + lax.axis_index("c")
    base = wid * _BPW
    pltpu.sync_copy(idx_hbm.at[pl.ds(base, _CHUNK)], idx_a)
    pltpu.sync_copy(idx_hbm.at[pl.ds(base + _CHUNK, _CHUNK)], idx_b)
    ca = pltpu.async_copy(cb_hbm.at[idx_a], rows_a, sem)
    cb = pltpu.async_copy(cb_hbm.at[idx_b], rows_b, sem)
    ca.wait()
    cb.wait()
    pltpu.sync_copy(rows_a, out_hbm.at[pl.ds(base, _CHUNK)])
    pltpu.sync_copy(rows_b, out_hbm.at[pl.ds(base + _CHUNK, _CHUNK)])


@functools.cache
def _sc_gather():
    mesh = plsc.VectorSubcoreMesh(core_axis_name="c", subcore_axis_name="s")
    return pl.kernel(
        _sc_gather_body,
        mesh=mesh,
        out_type=jax.ShapeDtypeStruct((B * T, 128), jnp.float32),
        scratch_types=[
            pltpu.VMEM((_CHUNK,), jnp.int32),
            pltpu.VMEM((_CHUNK,), jnp.int32),
            pltpu.VMEM((_CHUNK, 128), jnp.float32),
            pltpu.VMEM((_CHUNK, 128), jnp.float32),
            pltpu.SemaphoreType.DMA,
        ],
    )


def kernel(z, codebook, in_v, in_g, in_b, out_v, out_g, out_b):
    f32 = jnp.float32
    bf16 = jnp.bfloat16
    w_in = in_g * in_v / jnp.maximum(
        jnp.sqrt(jnp.sum(in_v * in_v, axis=1, keepdims=True)), 1e-12)
    w_out = out_g * out_v / jnp.maximum(
        jnp.sqrt(jnp.sum(out_v * out_v, axis=1, keepdims=True)), 1e-12)
    cb_n = codebook / jnp.maximum(
        jnp.sqrt(jnp.sum(codebook * codebook, axis=1, keepdims=True)), 1e-12)
    c = jnp.sum(cb_n ** 2, axis=1, keepdims=True).T

    ze = _stageA(z, w_in.astype(bf16), in_b.reshape(DC, 1))

    enc = jnp.transpose(ze, (0, 2, 1)).reshape(B * T, DC)
    n = jnp.sqrt(jnp.sum(enc * enc, axis=-1, keepdims=True))
    encn = enc / jnp.maximum(n, 1e-12)
    r = jnp.sum(encn ** 2, axis=1, keepdims=True)

    idx3 = _stageB(encn, cb_n, r, c)
    idx_flat = idx3.reshape(B * T)

    cb_pad = jnp.concatenate(
        [codebook, jnp.zeros((K, 128 - DC), f32)], axis=1)
    zq_flat = _sc_gather()(cb_pad, idx_flat)
    zq = zq_flat.reshape(B, T, 128)

    out, loss3 = _stageD(zq, ze, w_out.astype(bf16), out_b.reshape(CIN, 1))
    loss = loss3[:, 0, 0]
    indices = idx_flat.reshape(B, T)
    return (out, loss, loss, indices, ze)


# speedup vs baseline: 1.1842x; 1.0733x over previous
import functools

import jax
import jax.numpy as jnp
from jax import lax
from jax.experimental import pallas as pl
from jax.experimental.pallas import tpu as pltpu
from jax.experimental.pallas import tpu_sc as plsc

B, CIN, T = 8, 768, 1024
K, DC = 8192, 64
TBLK = 2048
KBLK = 2048
NT = 1
NTB = (B * T) // TBLK
NK = K // KBLK


def _stageA_body(z_ref, w_ref, b_ref, ze_ref):
    zb = z_ref[0].astype(jnp.bfloat16)
    ze = lax.dot_general(w_ref[...], zb, (((1,), (0,)), ((), ())),
                         preferred_element_type=jnp.float32)
    ze_ref[0] = ze + b_ref[...]


_STAGEA_KW = dict(
    grid=(B,),
    in_specs=[
        pl.BlockSpec((1, CIN, T), lambda b: (b, 0, 0)),
        pl.BlockSpec((DC, CIN), lambda b: (0, 0)),
        pl.BlockSpec((DC, 1), lambda b: (0, 0)),
    ],
    out_specs=pl.BlockSpec((1, DC, T), lambda b: (b, 0, 0)),
    out_shape=jax.ShapeDtypeStruct((B, DC, T), jnp.float32),
)

_stageA = pl.pallas_call(_stageA_body, **_STAGEA_KW)


def _stageB_body(encn_ref, cbn_ref, r_ref, c_ref, idx_ref, best_ref, bidx_ref):
    k = pl.program_id(1)

    @pl.when(k == 0)
    def _():
        best_ref[...] = jnp.full((TBLK, 1), -jnp.inf, jnp.float32)
        bidx_ref[...] = jnp.zeros((TBLK, 1), jnp.int32)

    s = lax.dot_general(encn_ref[...].astype(jnp.bfloat16),
                        cbn_ref[pl.ds(k * KBLK, KBLK), :].astype(jnp.bfloat16),
                        (((1,), (1,)), ((), ())),
                        preferred_element_type=jnp.float32)
    dist = (r_ref[...] - 2.0 * s) + c_ref[...]
    minval = jnp.min(dist, axis=1, keepdims=True)
    fiota = lax.broadcasted_iota(jnp.int32, (TBLK, KBLK), 1).astype(jnp.float32)
    fidx = jnp.min(jnp.where(dist == minval, fiota, float(KBLK)),
                   axis=1, keepdims=True)
    minidx = fidx.astype(jnp.int32) + k * KBLK
    cand = -minval
    better = cand > best_ref[...]
    new_best = jnp.where(
        better, cand.astype(jnp.bfloat16).astype(jnp.float32), best_ref[...])
    new_bidx = jnp.where(better, minidx, bidx_ref[...])
    best_ref[...] = new_best
    bidx_ref[...] = new_bidx

    @pl.when(k == NK - 1)
    def _():
        idx_ref[0] = new_bidx


_STAGEB_KW = dict(
    grid=(NTB, NK),
    in_specs=[
        pl.BlockSpec((TBLK, DC), lambda t, k: (t, 0)),
        pl.BlockSpec((K, DC), lambda t, k: (0, 0)),
        pl.BlockSpec((TBLK, 1), lambda t, k: (t, 0)),
        pl.BlockSpec((1, KBLK), lambda t, k: (0, k)),
    ],
    out_specs=pl.BlockSpec((1, TBLK, 1), lambda t, k: (t, 0, 0)),
    out_shape=jax.ShapeDtypeStruct((NTB, TBLK, 1), jnp.int32),
    scratch_shapes=[
        pltpu.VMEM((TBLK, 1), jnp.float32),
        pltpu.VMEM((TBLK, 1), jnp.int32),
    ],
)

_stageB = pl.pallas_call(_stageB_body, **_STAGEB_KW)


def _stageD_body(zq_ref, ze_ref, w_ref, b_ref, out_ref, loss_ref):
    zq = zq_ref[0][:, :DC]
    o = lax.dot_general(w_ref[...], zq.astype(jnp.bfloat16),
                        (((1,), (1,)), ((), ())),
                        preferred_element_type=jnp.float32)
    out_ref[0] = o + b_ref[...]
    diff = ze_ref[0] - zq.T
    msq = jnp.sum(diff * diff) * (1.0 / (DC * T))
    loss_ref[...] = jnp.full((1, 1, 128), msq, jnp.float32)


_STAGED_KW = dict(
    grid=(B,),
    in_specs=[
        pl.BlockSpec((1, T, 128), lambda b: (b, 0, 0)),
        pl.BlockSpec((1, DC, T), lambda b: (b, 0, 0)),
        pl.BlockSpec((CIN, DC), lambda b: (0, 0)),
        pl.BlockSpec((CIN, 1), lambda b: (0, 0)),
    ],
    out_specs=[
        pl.BlockSpec((1, CIN, T), lambda b: (b, 0, 0)),
        pl.BlockSpec((1, 1, 128), lambda b: (b, 0, 0)),
    ],
    out_shape=[
        jax.ShapeDtypeStruct((B, CIN, T), jnp.float32),
        jax.ShapeDtypeStruct((B, 1, 128), jnp.float32),
    ],
)

_stageD = pl.pallas_call(_stageD_body, **_STAGED_KW)

_NC, _NS = 2, 16
_NW = _NC * _NS
_BPW = (B * T) // _NW
_CHUNK = 128


def _sc_gather_body(cb_hbm, idx_hbm, out_hbm, idx_a, idx_b, rows_a, rows_b,
                    sem):
    wid = lax.axis_index("s") * _NC + lax.axis_index("c")
    base = wid * _BPW
    pltpu.sync_copy(idx_hbm.at[pl.ds(base, _CHUNK)], idx_a)
    pltpu.sync_copy(idx_hbm.at[pl.ds(base + _CHUNK, _CHUNK)], idx_b)
    ca = pltpu.async_copy(cb_hbm.at[idx_a], rows_a, sem)
    cb = pltpu.async_copy(cb_hbm.at[idx_b], rows_b, sem)
    ca.wait()
    cb.wait()
    pltpu.sync_copy(rows_a, out_hbm.at[pl.ds(base, _CHUNK)])
    pltpu.sync_copy(rows_b, out_hbm.at[pl.ds(base + _CHUNK, _CHUNK)])


@functools.cache
def _sc_gather():
    mesh = plsc.VectorSubcoreMesh(core_axis_name="c", subcore_axis_name="s")
    return pl.kernel(
        _sc_gather_body,
        mesh=mesh,
        out_type=jax.ShapeDtypeStruct((B * T, 128), jnp.float32),
        scratch_types=[
            pltpu.VMEM((_CHUNK,), jnp.int32),
            pltpu.VMEM((_CHUNK,), jnp.int32),
            pltpu.VMEM((_CHUNK, 128), jnp.float32),
            pltpu.VMEM((_CHUNK, 128), jnp.float32),
            pltpu.SemaphoreType.DMA,
        ],
    )


def kernel(z, codebook, in_v, in_g, in_b, out_v, out_g, out_b):
    f32 = jnp.float32
    bf16 = jnp.bfloat16
    w_in = in_g * in_v / jnp.maximum(
        jnp.sqrt(jnp.sum(in_v * in_v, axis=1, keepdims=True)), 1e-12)
    w_out = out_g * out_v / jnp.maximum(
        jnp.sqrt(jnp.sum(out_v * out_v, axis=1, keepdims=True)), 1e-12)
    cb_n = codebook / jnp.maximum(
        jnp.sqrt(jnp.sum(codebook * codebook, axis=1, keepdims=True)), 1e-12)
    c = jnp.sum(cb_n ** 2, axis=1, keepdims=True).T

    ze = _stageA(z, w_in.astype(bf16), in_b.reshape(DC, 1))

    enc = jnp.transpose(ze, (0, 2, 1)).reshape(B * T, DC)
    n = jnp.sqrt(jnp.sum(enc * enc, axis=-1, keepdims=True))
    encn = enc / jnp.maximum(n, 1e-12)
    r = jnp.sum(encn ** 2, axis=1, keepdims=True)

    idx3 = _stageB(encn, cb_n, r, c)
    idx_flat = idx3.reshape(B * T)

    cb_pad = jnp.concatenate(
        [codebook, jnp.zeros((K, 128 - DC), f32)], axis=1)
    zq_flat = _sc_gather()(cb_pad, idx_flat)
    zq = zq_flat.reshape(B, T, 128)

    out, loss3 = _stageD(zq, ze, w_out.astype(bf16), out_b.reshape(CIN, 1))
    loss = loss3[:, 0, 0]
    indices = idx_flat.reshape(B, T)
    return (out, loss, loss, indices, ze)
